# baseline (speedup 1.0000x reference)
import functools

import jax
import jax.numpy as jnp
from jax import lax
from jax.experimental import pallas as pl
from jax.experimental.pallas import tpu as pltpu
from jax.experimental.pallas import tpu_sc as plsc

_N = 10000
_E = 320000
_F_IN = 128
_H = 32
_G = 64

_NCORE = 2
_NSUB = 16
_NW = _NCORE * _NSUB
_CHUNK = 128
_NCT = 160
_NC0 = 104
_EPAD = _NSUB * _NCT * _CHUNK
_NPAD = 10112
_RPT = _NPAD // _NSUB

_mesh = plsc.VectorSubcoreMesh(core_axis_name="c", subcore_axis_name="s")


@functools.partial(
    pl.kernel,
    out_type=jax.ShapeDtypeStruct((_NCORE, _NPAD, _H), jnp.float32),
    mesh=_mesh,
    scratch_types=[
        pltpu.VMEM((_NCT, _CHUNK), jnp.int32),
        pltpu.VMEM((_NCT, _CHUNK), jnp.int32),
        pltpu.VMEM((4, _CHUNK, _H), jnp.float32),
        pltpu.VMEM((_RPT, _H), jnp.float32),
        pltpu.VMEM_SHARED((_NPAD, _H), jnp.float32),
        [pltpu.SemaphoreType.DMA] * 4,
    ],
    compiler_params=pltpu.CompilerParams(use_tc_tiling_on_sc=False),
)
def _edge_scatter_add(table_hbm, src_hbm, dst_hbm, out_hbm,
                      src_v, dst_v, rows_v, zbuf, acc_sh, sems):
    cid = lax.axis_index("c")
    sid = lax.axis_index("s")
    r0 = sid * _RPT
    lo = cid * _NC0
    hi = lo + _NC0 + cid * (_NCT - 2 * _NC0)

    z16 = jnp.zeros((16,), jnp.float32)

    @pl.loop(0, _RPT, unroll=8)
    def _(i):
        zbuf[i, pl.ds(0, 16)] = z16
        zbuf[i, pl.ds(16, 16)] = z16

    pltpu.sync_copy(zbuf, acc_sh.at[pl.ds(r0, _RPT)])
    @pl.when(cid == 0)
    def _():
        pltpu.sync_copy(src_hbm.at[sid, pl.ds(0, _NC0)],
                        src_v.at[pl.ds(0, _NC0)])
        pltpu.sync_copy(dst_hbm.at[sid, pl.ds(0, _NC0)],
                        dst_v.at[pl.ds(0, _NC0)])

    @pl.when(cid == 1)
    def _():
        pltpu.sync_copy(src_hbm.at[sid, pl.ds(_NC0, _NCT - _NC0)],
                        src_v.at[pl.ds(_NC0, _NCT - _NC0)])
        pltpu.sync_copy(dst_hbm.at[sid, pl.ds(_NC0, _NCT - _NC0)],
                        dst_v.at[pl.ds(_NC0, _NCT - _NC0)])

    plsc.subcore_barrier()

    for b in range(4):
        pltpu.async_copy(table_hbm.at[src_v.at[lo + b]], rows_v.at[b],
                         sems[b])

    @pl.loop(lo, hi, step=4)
    def _(i):
        for b in range(4):
            c = i + b
            buf = rows_v.at[b]
            pltpu.make_async_copy(table_hbm.at[src_v.at[c]], buf,
                                  sems[b]).wait()
            pltpu.sync_copy(buf, acc_sh.at[dst_v.at[c]], add=True)
            nxt = c + 4
            @pl.when(nxt < hi)
            def _():
                pltpu.async_copy(table_hbm.at[src_v.at[nxt]], buf, sems[b])

    plsc.subcore_barrier()
    pltpu.sync_copy(acc_sh.at[pl.ds(r0, _RPT)],
                    out_hbm.at[cid, pl.ds(r0, _RPT)])


_DW = 16


@functools.partial(
    pl.kernel,
    out_type=jax.ShapeDtypeStruct((_NCORE, _NPAD, _DW), jnp.float32),
    mesh=_mesh,
    scratch_types=[
        pltpu.VMEM((_NCT, _CHUNK), jnp.int32),
        pltpu.VMEM((_CHUNK, _DW), jnp.float32),
        pltpu.VMEM((_RPT, _DW), jnp.float32),
        pltpu.VMEM_SHARED((_NPAD, _DW), jnp.float32),
        pltpu.SemaphoreType.DMA,
    ],
    compiler_params=pltpu.CompilerParams(use_tc_tiling_on_sc=False),
)
def _degree_scatter(ones_hbm, dst_hbm, out_hbm,
                    dst_v, ones_v, zbuf, acc_sh, dsem):
    cid = lax.axis_index("c")
    sid = lax.axis_index("s")
    r0 = sid * _RPT
    lo = cid * _NC0
    hi = lo + _NC0 + cid * (_NCT - 2 * _NC0)

    z16 = jnp.zeros((16,), jnp.float32)

    @pl.loop(0, _RPT, unroll=8)
    def _(i):
        zbuf[i, pl.ds(0, 16)] = z16

    pltpu.sync_copy(zbuf, acc_sh.at[pl.ds(r0, _RPT)])

    @pl.when(cid == 0)
    def _():
        pltpu.sync_copy(dst_hbm.at[sid, pl.ds(0, _NC0)],
                        dst_v.at[pl.ds(0, _NC0)])

    @pl.when(cid == 1)
    def _():
        pltpu.sync_copy(dst_hbm.at[sid, pl.ds(_NC0, _NCT - _NC0)],
                        dst_v.at[pl.ds(_NC0, _NCT - _NC0)])

    pltpu.sync_copy(ones_hbm, ones_v)
    plsc.subcore_barrier()

    @pl.loop(lo, hi)
    def _(c):
        pltpu.async_copy(ones_v, acc_sh.at[dst_v.at[c]], dsem, add=True)

    @pl.loop(lo, hi)
    def _(c):
        pltpu.make_async_copy(ones_v, acc_sh.at[dst_v.at[c]], dsem).wait()

    plsc.subcore_barrier()
    pltpu.sync_copy(acc_sh.at[pl.ds(r0, _RPT)],
                    out_hbm.at[cid, pl.ds(r0, _RPT)])


def _tc0_body(x_ref, w_ref, degp_ref, dinv_ref, t_ref):
    deg = degp_ref[0] + degp_ref[1] + 1.0
    dinv16 = lax.rsqrt(jnp.maximum(deg, 1.0))
    dinv = jnp.concatenate((dinv16, dinv16), axis=1)
    dinv_ref[...] = dinv
    hw = jnp.dot(x_ref[...], w_ref[...], preferred_element_type=jnp.float32)
    t_ref[...] = dinv * hw


def _tc_mid_body(acc_ref, t_ref, dinv_ref, b_ref, w_ref, tn_ref):
    dinv = dinv_ref[...]
    h = dinv * (acc_ref[0] + acc_ref[1] + t_ref[...]) + b_ref[...]
    h = jnp.where(h >= 0, h, 0.01 * h)
    hw = jnp.dot(h, w_ref[...], preferred_element_type=jnp.float32)
    tn_ref[...] = dinv * hw


def _tc_fin_body(acc_ref, t_ref, dinv_ref, b_ref, batch_ref, wout_ref,
                 bout_ref, hid_ref, out_ref, pooled_ref):
    h = dinv_ref[...] * (acc_ref[0] + acc_ref[1] + t_ref[...]) + b_ref[...]
    h = jnp.where(h >= 0, h, 0.01 * h)
    hid_ref[...] = h
    hn = h[:_N]
    bi = batch_ref[...]
    neg_inf = jnp.float32(float("-inf"))

    def seg(g, _):
        m = jnp.where(bi == g, hn, neg_inf)
        pooled_ref[pl.ds(g, 1), :] = jnp.max(m, axis=0)[None, :]
        return 0

    lax.fori_loop(0, _G, seg, 0)
    pooled = pooled_ref[...]
    out_ref[...] = (
        jnp.dot(pooled, wout_ref[...], preferred_element_type=jnp.float32)
        + bout_ref[...]
    )


_tc0 = pl.pallas_call(
    _tc0_body,
    out_shape=[
        jax.ShapeDtypeStruct((_NPAD, _H), jnp.float32),
        jax.ShapeDtypeStruct((_NPAD, _H), jnp.float32),
    ],
)

_tc_mid = pl.pallas_call(
    _tc_mid_body,
    out_shape=jax.ShapeDtypeStruct((_NPAD, _H), jnp.float32),
)

_tc_fin = pl.pallas_call(
    _tc_fin_body,
    out_shape=[
        jax.ShapeDtypeStruct((_NPAD, _H), jnp.float32),
        jax.ShapeDtypeStruct((_G, 1), jnp.float32),
    ],
    scratch_shapes=[pltpu.VMEM((_G, _H), jnp.float32)],
)


def kernel(x, edge_index, batch_index,
           W0, b0, W1, b1, W2, b2, W3, b3, W_out, b_out):
    src = edge_index[0]
    dst = edge_index[1]
    pad = _EPAD - _E
    srcp = jnp.concatenate(
        [src, jnp.zeros((pad,), jnp.int32)]).reshape(_NSUB, _NCT, _CHUNK)
    dstp = jnp.concatenate(
        [dst, jnp.full((pad,), _N, jnp.int32)]).reshape(_NSUB, _NCT, _CHUNK)
    ones = jnp.ones((_CHUNK, _DW), jnp.float32)
    xp = jnp.pad(x, ((0, _NPAD - _N), (0, 0)))
    batch32 = jnp.broadcast_to(batch_index[:, None], (_N, _H))

    degp = _degree_scatter(ones, dstp)
    dinv, t = _tc0(xp, W0, degp)
    for b, Wn in ((b0, W1), (b1, W2), (b2, W3)):
        acc = _edge_scatter_add(t, srcp, dstp)
        t = _tc_mid(acc, t, dinv, b.reshape(1, _H), Wn)
    acc = _edge_scatter_add(t, srcp, dstp)
    hid_pad, out = _tc_fin(acc, t, dinv, b3.reshape(1, _H), batch32,
                           W_out, b_out.reshape(1, 1))
    return (out, hid_pad[:_N])


# speedup vs baseline: 1.5506x; 1.5506x over previous
import functools

import jax
import jax.numpy as jnp
from jax import lax
from jax.experimental import pallas as pl
from jax.experimental.pallas import tpu as pltpu
from jax.experimental.pallas import tpu_sc as plsc

_N = 10000
_E = 320000
_F_IN = 128
_H = 32
_G = 64

_NCORE = 2
_NSUB = 16
_NW = _NCORE * _NSUB
_CHUNK = 128
_NCT = 160
_NC0 = 104
_EPAD = _NSUB * _NCT * _CHUNK
_NPAD = 10112
_RPT = _NPAD // _NSUB

_mesh = plsc.VectorSubcoreMesh(core_axis_name="c", subcore_axis_name="s")


@functools.partial(
    pl.kernel,
    out_type=jax.ShapeDtypeStruct((_NCORE, _NPAD, _H), jnp.float32),
    mesh=_mesh,
    scratch_types=[
        pltpu.VMEM((_NCT, _CHUNK), jnp.int32),
        pltpu.VMEM((_NCT, _CHUNK), jnp.int32),
        pltpu.VMEM((4, _CHUNK, _H), jnp.float32),
        pltpu.VMEM((_RPT, _H), jnp.float32),
        pltpu.VMEM_SHARED((_NPAD, _H), jnp.float32),
        pltpu.VMEM_SHARED((_NPAD, _H), jnp.float32),
        [pltpu.SemaphoreType.DMA] * 4,
    ],
    compiler_params=pltpu.CompilerParams(use_tc_tiling_on_sc=False),
)
def _edge_scatter_add(table_hbm, src_hbm, dst_hbm, out_hbm,
                      src_v, dst_v, rows_v, zbuf, acc_sh, table_sh, sems):
    cid = lax.axis_index("c")
    sid = lax.axis_index("s")
    r0 = sid * _RPT
    lo = cid * _NC0
    hi = lo + _NC0 + cid * (_NCT - 2 * _NC0)

    z16 = jnp.zeros((16,), jnp.float32)

    @pl.loop(0, _RPT, unroll=8)
    def _(i):
        zbuf[i, pl.ds(0, 16)] = z16
        zbuf[i, pl.ds(16, 16)] = z16

    pltpu.sync_copy(zbuf, acc_sh.at[pl.ds(r0, _RPT)])
    pltpu.sync_copy(table_hbm.at[pl.ds(r0, _RPT)],
                    table_sh.at[pl.ds(r0, _RPT)])
    @pl.when(cid == 0)
    def _():
        pltpu.sync_copy(src_hbm.at[sid, pl.ds(0, _NC0)],
                        src_v.at[pl.ds(0, _NC0)])
        pltpu.sync_copy(dst_hbm.at[sid, pl.ds(0, _NC0)],
                        dst_v.at[pl.ds(0, _NC0)])

    @pl.when(cid == 1)
    def _():
        pltpu.sync_copy(src_hbm.at[sid, pl.ds(_NC0, _NCT - _NC0)],
                        src_v.at[pl.ds(_NC0, _NCT - _NC0)])
        pltpu.sync_copy(dst_hbm.at[sid, pl.ds(_NC0, _NCT - _NC0)],
                        dst_v.at[pl.ds(_NC0, _NCT - _NC0)])

    plsc.subcore_barrier()

    for b in range(4):
        pltpu.async_copy(table_sh.at[src_v.at[lo + b]], rows_v.at[b],
                         sems[b])

    @pl.loop(lo, hi, step=4)
    def _(i):
        for b in range(4):
            c = i + b
            buf = rows_v.at[b]
            pltpu.make_async_copy(table_sh.at[src_v.at[c]], buf,
                                  sems[b]).wait()
            pltpu.sync_copy(buf, acc_sh.at[dst_v.at[c]], add=True)
            nxt = c + 4
            @pl.when(nxt < hi)
            def _():
                pltpu.async_copy(table_sh.at[src_v.at[nxt]], buf, sems[b])

    plsc.subcore_barrier()
    pltpu.sync_copy(acc_sh.at[pl.ds(r0, _RPT)],
                    out_hbm.at[cid, pl.ds(r0, _RPT)])


_DW = 16


@functools.partial(
    pl.kernel,
    out_type=jax.ShapeDtypeStruct((_NCORE, _NPAD, _DW), jnp.float32),
    mesh=_mesh,
    scratch_types=[
        pltpu.VMEM((_NCT, _CHUNK), jnp.int32),
        pltpu.VMEM((_CHUNK, _DW), jnp.float32),
        pltpu.VMEM((_RPT, _DW), jnp.float32),
        pltpu.VMEM_SHARED((_NPAD, _DW), jnp.float32),
        pltpu.SemaphoreType.DMA,
    ],
    compiler_params=pltpu.CompilerParams(use_tc_tiling_on_sc=False),
)
def _degree_scatter(ones_hbm, dst_hbm, out_hbm,
                    dst_v, ones_v, zbuf, acc_sh, dsem):
    cid = lax.axis_index("c")
    sid = lax.axis_index("s")
    r0 = sid * _RPT
    lo = cid * _NC0
    hi = lo + _NC0 + cid * (_NCT - 2 * _NC0)

    z16 = jnp.zeros((16,), jnp.float32)

    @pl.loop(0, _RPT, unroll=8)
    def _(i):
        zbuf[i, pl.ds(0, 16)] = z16

    pltpu.sync_copy(zbuf, acc_sh.at[pl.ds(r0, _RPT)])

    @pl.when(cid == 0)
    def _():
        pltpu.sync_copy(dst_hbm.at[sid, pl.ds(0, _NC0)],
                        dst_v.at[pl.ds(0, _NC0)])

    @pl.when(cid == 1)
    def _():
        pltpu.sync_copy(dst_hbm.at[sid, pl.ds(_NC0, _NCT - _NC0)],
                        dst_v.at[pl.ds(_NC0, _NCT - _NC0)])

    pltpu.sync_copy(ones_hbm, ones_v)
    plsc.subcore_barrier()

    @pl.loop(lo, hi)
    def _(c):
        pltpu.async_copy(ones_v, acc_sh.at[dst_v.at[c]], dsem, add=True)

    @pl.loop(lo, hi)
    def _(c):
        pltpu.make_async_copy(ones_v, acc_sh.at[dst_v.at[c]], dsem).wait()

    plsc.subcore_barrier()
    pltpu.sync_copy(acc_sh.at[pl.ds(r0, _RPT)],
                    out_hbm.at[cid, pl.ds(r0, _RPT)])


def _tc0_body(x_ref, w_ref, degp_ref, dinv_ref, t_ref):
    deg = degp_ref[0] + degp_ref[1] + 1.0
    dinv16 = lax.rsqrt(jnp.maximum(deg, 1.0))
    dinv = jnp.concatenate((dinv16, dinv16), axis=1)
    dinv_ref[...] = dinv
    hw = jnp.dot(x_ref[...], w_ref[...], preferred_element_type=jnp.float32)
    t_ref[...] = dinv * hw


def _tc_mid_body(acc_ref, t_ref, dinv_ref, b_ref, w_ref, tn_ref):
    dinv = dinv_ref[...]
    h = dinv * (acc_ref[0] + acc_ref[1] + t_ref[...]) + b_ref[...]
    h = jnp.where(h >= 0, h, 0.01 * h)
    hw = jnp.dot(h, w_ref[...], preferred_element_type=jnp.float32)
    tn_ref[...] = dinv * hw


def _tc_fin_body(acc_ref, t_ref, dinv_ref, b_ref, batch_ref, wout_ref,
                 bout_ref, hid_ref, out_ref, pooled_ref):
    h = dinv_ref[...] * (acc_ref[0] + acc_ref[1] + t_ref[...]) + b_ref[...]
    h = jnp.where(h >= 0, h, 0.01 * h)
    hid_ref[...] = h
    hn = h[:_N]
    bi = batch_ref[...]
    neg_inf = jnp.float32(float("-inf"))

    def seg(g, _):
        m = jnp.where(bi == g, hn, neg_inf)
        pooled_ref[pl.ds(g, 1), :] = jnp.max(m, axis=0)[None, :]
        return 0

    lax.fori_loop(0, _G, seg, 0)
    pooled = pooled_ref[...]
    out_ref[...] = (
        jnp.dot(pooled, wout_ref[...], preferred_element_type=jnp.float32)
        + bout_ref[...]
    )


_tc0 = pl.pallas_call(
    _tc0_body,
    out_shape=[
        jax.ShapeDtypeStruct((_NPAD, _H), jnp.float32),
        jax.ShapeDtypeStruct((_NPAD, _H), jnp.float32),
    ],
)

_tc_mid = pl.pallas_call(
    _tc_mid_body,
    out_shape=jax.ShapeDtypeStruct((_NPAD, _H), jnp.float32),
)

_tc_fin = pl.pallas_call(
    _tc_fin_body,
    out_shape=[
        jax.ShapeDtypeStruct((_NPAD, _H), jnp.float32),
        jax.ShapeDtypeStruct((_G, 1), jnp.float32),
    ],
    scratch_shapes=[pltpu.VMEM((_G, _H), jnp.float32)],
)


def kernel(x, edge_index, batch_index,
           W0, b0, W1, b1, W2, b2, W3, b3, W_out, b_out):
    src = edge_index[0]
    dst = edge_index[1]
    pad = _EPAD - _E
    srcp = jnp.concatenate(
        [src, jnp.zeros((pad,), jnp.int32)]).reshape(_NSUB, _NCT, _CHUNK)
    dstp = jnp.concatenate(
        [dst, jnp.full((pad,), _N, jnp.int32)]).reshape(_NSUB, _NCT, _CHUNK)
    ones = jnp.ones((_CHUNK, _DW), jnp.float32)
    xp = jnp.pad(x, ((0, _NPAD - _N), (0, 0)))
    batch32 = jnp.broadcast_to(batch_index[:, None], (_N, _H))

    degp = _degree_scatter(ones, dstp)
    dinv, t = _tc0(xp, W0, degp)
    for b, Wn in ((b0, W1), (b1, W2), (b2, W3)):
        acc = _edge_scatter_add(t, srcp, dstp)
        t = _tc_mid(acc, t, dinv, b.reshape(1, _H), Wn)
    acc = _edge_scatter_add(t, srcp, dstp)
    hid_pad, out = _tc_fin(acc, t, dinv, b3.reshape(1, _H), batch32,
                           W_out, b_out.reshape(1, 1))
    return (out, hid_pad[:_N])


# speedup vs baseline: 1.9932x; 1.2855x over previous
import functools

import jax
import jax.numpy as jnp
from jax import lax
from jax.experimental import pallas as pl
from jax.experimental.pallas import tpu as pltpu
from jax.experimental.pallas import tpu_sc as plsc

_N = 10000
_E = 320000
_F_IN = 128
_H = 32
_G = 64

_NCORE = 2
_NSUB = 16
_NW = _NCORE * _NSUB
_CHUNK = 128
_NCT = 160
_NC0 = 84
_EPAD = _NSUB * _NCT * _CHUNK
_NPAD = 10240
_RPT = _NPAD // _NSUB
_RPW = _NPAD // _NW

_mesh = plsc.VectorSubcoreMesh(core_axis_name="c", subcore_axis_name="s")


@functools.partial(
    pl.kernel,
    out_type=jax.ShapeDtypeStruct((_NCORE, _NPAD, _H), jnp.float32),
    mesh=_mesh,
    scratch_types=[
        pltpu.VMEM((_NCT, _CHUNK), jnp.int32),
        pltpu.VMEM((_NCT, _CHUNK), jnp.int32),
        pltpu.VMEM((4, _CHUNK, _H), jnp.float32),
        pltpu.VMEM((_RPT, _H), jnp.float32),
        pltpu.VMEM_SHARED((_NPAD, _H), jnp.float32),
        pltpu.VMEM_SHARED((_NPAD, _H), jnp.float32),
        [pltpu.SemaphoreType.DMA] * 4,
    ],
    compiler_params=pltpu.CompilerParams(use_tc_tiling_on_sc=False),
)
def _edge_scatter_add(table_hbm, src_hbm, dst_hbm, out_hbm,
                      src_v, dst_v, rows_v, zbuf, acc_sh, table_sh, sems):
    cid = lax.axis_index("c")
    sid = lax.axis_index("s")
    r0 = sid * _RPT
    lo = cid * _NC0
    hi = lo + _NC0 + cid * (_NCT - 2 * _NC0)

    z16 = jnp.zeros((16,), jnp.float32)

    @pl.loop(0, _RPT, unroll=8)
    def _(i):
        zbuf[i, pl.ds(0, 16)] = z16
        zbuf[i, pl.ds(16, 16)] = z16

    pltpu.sync_copy(zbuf, acc_sh.at[pl.ds(r0, _RPT)])
    pltpu.sync_copy(table_hbm.at[pl.ds(r0, _RPT)],
                    table_sh.at[pl.ds(r0, _RPT)])
    @pl.when(cid == 0)
    def _():
        pltpu.sync_copy(src_hbm.at[sid, pl.ds(0, _NC0)],
                        src_v.at[pl.ds(0, _NC0)])
        pltpu.sync_copy(dst_hbm.at[sid, pl.ds(0, _NC0)],
                        dst_v.at[pl.ds(0, _NC0)])

    @pl.when(cid == 1)
    def _():
        pltpu.sync_copy(src_hbm.at[sid, pl.ds(_NC0, _NCT - _NC0)],
                        src_v.at[pl.ds(_NC0, _NCT - _NC0)])
        pltpu.sync_copy(dst_hbm.at[sid, pl.ds(_NC0, _NCT - _NC0)],
                        dst_v.at[pl.ds(_NC0, _NCT - _NC0)])

    plsc.subcore_barrier()

    for b in range(4):
        pltpu.async_copy(table_sh.at[src_v.at[lo + b]], rows_v.at[b],
                         sems[b])

    @pl.loop(lo, hi, step=4)
    def _(i):
        for b in range(4):
            c = i + b
            buf = rows_v.at[b]
            pltpu.make_async_copy(table_sh.at[src_v.at[c]], buf,
                                  sems[b]).wait()
            pltpu.sync_copy(buf, acc_sh.at[dst_v.at[c]], add=True)
            nxt = c + 4
            @pl.when(nxt < hi)
            def _():
                pltpu.async_copy(table_sh.at[src_v.at[nxt]], buf, sems[b])

    plsc.subcore_barrier()
    pltpu.sync_copy(acc_sh.at[pl.ds(r0, _RPT)],
                    out_hbm.at[cid, pl.ds(r0, _RPT)])


_DW = 16


@functools.partial(
    pl.kernel,
    out_type=jax.ShapeDtypeStruct((_NCORE, _NPAD, _DW), jnp.float32),
    mesh=_mesh,
    scratch_types=[
        pltpu.VMEM((_NCT, _CHUNK), jnp.int32),
        pltpu.VMEM((_CHUNK, _DW), jnp.float32),
        pltpu.VMEM((_RPT, _DW), jnp.float32),
        pltpu.VMEM_SHARED((_NPAD, _DW), jnp.float32),
        pltpu.SemaphoreType.DMA,
    ],
    compiler_params=pltpu.CompilerParams(use_tc_tiling_on_sc=False),
)
def _degree_scatter(ones_hbm, dst_hbm, out_hbm,
                    dst_v, ones_v, zbuf, acc_sh, dsem):
    cid = lax.axis_index("c")
    sid = lax.axis_index("s")
    r0 = sid * _RPT
    lo = cid * _NC0
    hi = lo + _NC0 + cid * (_NCT - 2 * _NC0)

    z16 = jnp.zeros((16,), jnp.float32)

    @pl.loop(0, _RPT, unroll=8)
    def _(i):
        zbuf[i, pl.ds(0, 16)] = z16

    pltpu.sync_copy(zbuf, acc_sh.at[pl.ds(r0, _RPT)])

    @pl.when(cid == 0)
    def _():
        pltpu.sync_copy(dst_hbm.at[sid, pl.ds(0, _NC0)],
                        dst_v.at[pl.ds(0, _NC0)])

    @pl.when(cid == 1)
    def _():
        pltpu.sync_copy(dst_hbm.at[sid, pl.ds(_NC0, _NCT - _NC0)],
                        dst_v.at[pl.ds(_NC0, _NCT - _NC0)])

    pltpu.sync_copy(ones_hbm, ones_v)
    plsc.subcore_barrier()

    @pl.loop(lo, hi)
    def _(c):
        pltpu.async_copy(ones_v, acc_sh.at[dst_v.at[c]], dsem, add=True)

    @pl.loop(lo, hi)
    def _(c):
        pltpu.make_async_copy(ones_v, acc_sh.at[dst_v.at[c]], dsem).wait()

    plsc.subcore_barrier()
    pltpu.sync_copy(acc_sh.at[pl.ds(r0, _RPT)],
                    out_hbm.at[cid, pl.ds(r0, _RPT)])


@functools.partial(
    pl.kernel,
    out_type=[
        jax.ShapeDtypeStruct((_NPAD, _H), jnp.float32),
        jax.ShapeDtypeStruct((_NW, _G + 1, _H), jnp.float32),
    ],
    mesh=_mesh,
    scratch_types=[
        pltpu.VMEM((_RPW, _H), jnp.float32),
        pltpu.VMEM((_RPW, _H), jnp.float32),
        pltpu.VMEM((_RPW, _H), jnp.float32),
        pltpu.VMEM((_RPW, _H), jnp.float32),
        pltpu.VMEM((_RPW, _H), jnp.float32),
        pltpu.VMEM((_RPW,), jnp.int32),
        pltpu.VMEM((1, _H), jnp.float32),
        pltpu.VMEM((_G + 1, _H), jnp.float32),
        pltpu.SemaphoreType.DMA,
    ],
    compiler_params=pltpu.CompilerParams(use_tc_tiling_on_sc=False),
)
def _final_pool(acc_hbm, t_hbm, dinv_hbm, b_hbm, batch_hbm, h_hbm, part_hbm,
                a0_v, a1_v, t_v, dinv_v, h_v, batch_v, b_v, res_v, sem):
    cid = lax.axis_index("c")
    sid = lax.axis_index("s")
    wid = cid * _NSUB + sid
    r0 = wid * _RPW

    rows = pl.ds(r0, _RPW)
    copies = [
        pltpu.async_copy(acc_hbm.at[0, rows], a0_v, sem),
        pltpu.async_copy(acc_hbm.at[1, rows], a1_v, sem),
        pltpu.async_copy(t_hbm.at[rows], t_v, sem),
        pltpu.async_copy(dinv_hbm.at[rows], dinv_v, sem),
        pltpu.async_copy(batch_hbm.at[rows], batch_v, sem),
        pltpu.async_copy(b_hbm, b_v, sem),
    ]
    for c in copies:
        c.wait()

    neg = jnp.full((16,), float("-inf"), jnp.float32)

    @pl.loop(0, _G + 1)
    def _(g):
        res_v[g, pl.ds(0, 16)] = neg
        res_v[g, pl.ds(16, 16)] = neg

    b_lo = b_v[0, pl.ds(0, 16)]
    b_hi = b_v[0, pl.ds(16, 16)]

    @pl.loop(0, _RPW // 16)
    def _(i):
        base = i * 16
        bvec = batch_v[pl.ds(base, 16)]
        for j in range(16):
            r = base + j
            g = bvec[j]
            for half, bh in ((0, b_lo), (16, b_hi)):
                ds = pl.ds(half, 16)
                v = (dinv_v[r, ds]
                     * (a0_v[r, ds] + a1_v[r, ds] + t_v[r, ds]) + bh)
                v = jnp.where(v >= 0, v, 0.01 * v)
                h_v[r, ds] = v
                res_v[g, ds] = jnp.maximum(res_v[g, ds], v)

    pltpu.sync_copy(h_v, h_hbm.at[rows])
    pltpu.sync_copy(res_v, part_hbm.at[wid])


def _tc0_body(x_ref, w_ref, degp_ref, dinv_ref, t_ref):
    deg = degp_ref[0] + degp_ref[1] + 1.0
    dinv16 = lax.rsqrt(jnp.maximum(deg, 1.0))
    dinv = jnp.concatenate((dinv16, dinv16), axis=1)
    dinv_ref[...] = dinv
    hw = jnp.dot(x_ref[...], w_ref[...], preferred_element_type=jnp.float32)
    t_ref[...] = dinv * hw


def _tc_mid_body(acc_ref, t_ref, dinv_ref, b_ref, w_ref, tn_ref):
    dinv = dinv_ref[...]
    h = dinv * (acc_ref[0] + acc_ref[1] + t_ref[...]) + b_ref[...]
    h = jnp.where(h >= 0, h, 0.01 * h)
    hw = jnp.dot(h, w_ref[...], preferred_element_type=jnp.float32)
    tn_ref[...] = dinv * hw


def _tc_tail_body(part_ref, wout_ref, bout_ref, out_ref):
    pooled = jnp.max(part_ref[...], axis=0)[:_G]
    out_ref[...] = (
        jnp.dot(pooled, wout_ref[...], preferred_element_type=jnp.float32)
        + bout_ref[...]
    )


_tc0 = pl.pallas_call(
    _tc0_body,
    out_shape=[
        jax.ShapeDtypeStruct((_NPAD, _H), jnp.float32),
        jax.ShapeDtypeStruct((_NPAD, _H), jnp.float32),
    ],
)

_tc_mid = pl.pallas_call(
    _tc_mid_body,
    out_shape=jax.ShapeDtypeStruct((_NPAD, _H), jnp.float32),
)

_tc_tail = pl.pallas_call(
    _tc_tail_body,
    out_shape=jax.ShapeDtypeStruct((_G, 1), jnp.float32),
)


def kernel(x, edge_index, batch_index,
           W0, b0, W1, b1, W2, b2, W3, b3, W_out, b_out):
    src = edge_index[0]
    dst = edge_index[1]
    pad = _EPAD - _E
    srcp = jnp.concatenate(
        [src, jnp.zeros((pad,), jnp.int32)]).reshape(_NSUB, _NCT, _CHUNK)
    dstp = jnp.concatenate(
        [dst, jnp.full((pad,), _N, jnp.int32)]).reshape(_NSUB, _NCT, _CHUNK)
    ones = jnp.ones((_CHUNK, _DW), jnp.float32)
    xp = jnp.pad(x, ((0, _NPAD - _N), (0, 0)))
    batchp = jnp.concatenate(
        [batch_index, jnp.full((_NPAD - _N,), _G, jnp.int32)])

    degp = _degree_scatter(ones, dstp)
    dinv, t = _tc0(xp, W0, degp)
    for b, Wn in ((b0, W1), (b1, W2), (b2, W3)):
        acc = _edge_scatter_add(t, srcp, dstp)
        t = _tc_mid(acc, t, dinv, b.reshape(1, _H), Wn)
    acc = _edge_scatter_add(t, srcp, dstp)
    hid_pad, parts = _final_pool(acc, t, dinv, b3.reshape(1, _H), batchp)
    out = _tc_tail(parts, W_out, b_out.reshape(1, 1))
    return (out, hid_pad[:_N])


# speedup vs baseline: 2.0970x; 1.0521x over previous
import functools

import jax
import jax.numpy as jnp
from jax import lax
from jax.experimental import pallas as pl
from jax.experimental.pallas import tpu as pltpu
from jax.experimental.pallas import tpu_sc as plsc

_N = 10000
_E = 320000
_F_IN = 128
_H = 32
_G = 64

_NCORE = 2
_NSUB = 16
_NW = _NCORE * _NSUB
_CHUNK = 128
_NCT = 160
_NC0 = 88
_EPAD = _NSUB * _NCT * _CHUNK
_NPAD = 10240
_RPT = _NPAD // _NSUB
_RPW = _NPAD // _NW

_mesh = plsc.VectorSubcoreMesh(core_axis_name="c", subcore_axis_name="s")


@functools.partial(
    pl.kernel,
    out_type=jax.ShapeDtypeStruct((_NCORE, _NPAD, _H), jnp.float32),
    mesh=_mesh,
    scratch_types=[
        pltpu.VMEM((_NCT, _CHUNK), jnp.int32),
        pltpu.VMEM((_NCT, _CHUNK), jnp.int32),
        pltpu.VMEM((8, _CHUNK, _H), jnp.float32),
        pltpu.VMEM((_RPT // 4, _H), jnp.float32),
        pltpu.VMEM_SHARED((_NPAD, _H), jnp.float32),
        pltpu.VMEM_SHARED((_NPAD, _H), jnp.float32),
        [pltpu.SemaphoreType.DMA] * 8,
        [pltpu.SemaphoreType.DMA] * 8,
    ],
    compiler_params=pltpu.CompilerParams(use_tc_tiling_on_sc=False),
)
def _edge_scatter_add(table_hbm, src_hbm, dst_hbm, out_hbm,
                      src_v, dst_v, rows_v, zbuf, acc_sh, table_sh, sems,
                      ssems):
    cid = lax.axis_index("c")
    sid = lax.axis_index("s")
    r0 = sid * _RPT
    lo = cid * _NC0
    hi = lo + _NC0 + cid * (_NCT - 2 * _NC0)

    z16 = jnp.zeros((16,), jnp.float32)

    @pl.loop(0, _RPT // 4, unroll=8)
    def _(i):
        zbuf[i, pl.ds(0, 16)] = z16
        zbuf[i, pl.ds(16, 16)] = z16

    for q in range(4):
        pltpu.sync_copy(zbuf,
                        acc_sh.at[pl.ds(r0 + q * (_RPT // 4), _RPT // 4)])
    pltpu.sync_copy(table_hbm.at[pl.ds(r0, _RPT)],
                    table_sh.at[pl.ds(r0, _RPT)])
    @pl.when(cid == 0)
    def _():
        pltpu.sync_copy(src_hbm.at[sid, pl.ds(0, _NC0)],
                        src_v.at[pl.ds(0, _NC0)])
        pltpu.sync_copy(dst_hbm.at[sid, pl.ds(0, _NC0)],
                        dst_v.at[pl.ds(0, _NC0)])

    @pl.when(cid == 1)
    def _():
        pltpu.sync_copy(src_hbm.at[sid, pl.ds(_NC0, _NCT - _NC0)],
                        src_v.at[pl.ds(_NC0, _NCT - _NC0)])
        pltpu.sync_copy(dst_hbm.at[sid, pl.ds(_NC0, _NCT - _NC0)],
                        dst_v.at[pl.ds(_NC0, _NCT - _NC0)])

    plsc.subcore_barrier()

    for b in range(4):
        pltpu.async_copy(table_sh.at[src_v.at[lo + b]], rows_v.at[b],
                         sems[b])

    @pl.loop(lo, hi, step=8)
    def _(i):
        for b in range(8):
            c = i + b
            buf = rows_v.at[b]
            pltpu.make_async_copy(table_sh.at[src_v.at[c]], buf,
                                  sems[b]).wait()
            pltpu.async_copy(buf, acc_sh.at[dst_v.at[c]], ssems[b], add=True)
            b4 = (b + 4) % 8
            cm4 = c - 4
            @pl.when(cm4 >= lo)
            def _():
                pltpu.make_async_copy(rows_v.at[b4],
                                      acc_sh.at[dst_v.at[cm4]],
                                      ssems[b4]).wait()
            cp4 = c + 4
            @pl.when(cp4 < hi)
            def _():
                pltpu.async_copy(table_sh.at[src_v.at[cp4]], rows_v.at[b4],
                                 sems[b4])

    for k in range(4):
        pltpu.make_async_copy(rows_v.at[4 + k],
                              acc_sh.at[dst_v.at[hi - 4 + k]],
                              ssems[4 + k]).wait()

    plsc.subcore_barrier()
    pltpu.sync_copy(acc_sh.at[pl.ds(r0, _RPT)],
                    out_hbm.at[cid, pl.ds(r0, _RPT)])


_DW = 16


@functools.partial(
    pl.kernel,
    out_type=jax.ShapeDtypeStruct((_NCORE, _NPAD, _DW), jnp.float32),
    mesh=_mesh,
    scratch_types=[
        pltpu.VMEM((_NCT, _CHUNK), jnp.int32),
        pltpu.VMEM((_CHUNK, _DW), jnp.float32),
        pltpu.VMEM((_RPT, _DW), jnp.float32),
        pltpu.VMEM_SHARED((_NPAD, _DW), jnp.float32),
        pltpu.SemaphoreType.DMA,
    ],
    compiler_params=pltpu.CompilerParams(use_tc_tiling_on_sc=False),
)
def _degree_scatter(ones_hbm, dst_hbm, out_hbm,
                    dst_v, ones_v, zbuf, acc_sh, dsem):
    cid = lax.axis_index("c")
    sid = lax.axis_index("s")
    r0 = sid * _RPT
    lo = cid * _NC0
    hi = lo + _NC0 + cid * (_NCT - 2 * _NC0)

    z16 = jnp.zeros((16,), jnp.float32)

    @pl.loop(0, _RPT, unroll=8)
    def _(i):
        zbuf[i, pl.ds(0, 16)] = z16

    pltpu.sync_copy(zbuf, acc_sh.at[pl.ds(r0, _RPT)])

    @pl.when(cid == 0)
    def _():
        pltpu.sync_copy(dst_hbm.at[sid, pl.ds(0, _NC0)],
                        dst_v.at[pl.ds(0, _NC0)])

    @pl.when(cid == 1)
    def _():
        pltpu.sync_copy(dst_hbm.at[sid, pl.ds(_NC0, _NCT - _NC0)],
                        dst_v.at[pl.ds(_NC0, _NCT - _NC0)])

    pltpu.sync_copy(ones_hbm, ones_v)
    plsc.subcore_barrier()

    @pl.loop(lo, hi)
    def _(c):
        pltpu.async_copy(ones_v, acc_sh.at[dst_v.at[c]], dsem, add=True)

    @pl.loop(lo, hi)
    def _(c):
        pltpu.make_async_copy(ones_v, acc_sh.at[dst_v.at[c]], dsem).wait()

    plsc.subcore_barrier()
    pltpu.sync_copy(acc_sh.at[pl.ds(r0, _RPT)],
                    out_hbm.at[cid, pl.ds(r0, _RPT)])


@functools.partial(
    pl.kernel,
    out_type=[
        jax.ShapeDtypeStruct((_NPAD, _H), jnp.float32),
        jax.ShapeDtypeStruct((_NW, _G + 1, _H), jnp.float32),
    ],
    mesh=_mesh,
    scratch_types=[
        pltpu.VMEM((_RPW, _H), jnp.float32),
        pltpu.VMEM((_RPW, _H), jnp.float32),
        pltpu.VMEM((_RPW, _H), jnp.float32),
        pltpu.VMEM((_RPW, _H), jnp.float32),
        pltpu.VMEM((_RPW, _H), jnp.float32),
        pltpu.VMEM((_RPW,), jnp.int32),
        pltpu.VMEM((1, _H), jnp.float32),
        pltpu.VMEM((_G + 1, _H), jnp.float32),
        pltpu.SemaphoreType.DMA,
    ],
    compiler_params=pltpu.CompilerParams(use_tc_tiling_on_sc=False),
)
def _final_pool(acc_hbm, t_hbm, dinv_hbm, b_hbm, batch_hbm, h_hbm, part_hbm,
                a0_v, a1_v, t_v, dinv_v, h_v, batch_v, b_v, res_v, sem):
    cid = lax.axis_index("c")
    sid = lax.axis_index("s")
    wid = cid * _NSUB + sid
    r0 = wid * _RPW

    rows = pl.ds(r0, _RPW)
    copies = [
        pltpu.async_copy(acc_hbm.at[0, rows], a0_v, sem),
        pltpu.async_copy(acc_hbm.at[1, rows], a1_v, sem),
        pltpu.async_copy(t_hbm.at[rows], t_v, sem),
        pltpu.async_copy(dinv_hbm.at[rows], dinv_v, sem),
        pltpu.async_copy(batch_hbm.at[rows], batch_v, sem),
        pltpu.async_copy(b_hbm, b_v, sem),
    ]
    for c in copies:
        c.wait()

    neg = jnp.full((16,), float("-inf"), jnp.float32)

    @pl.loop(0, _G + 1)
    def _(g):
        res_v[g, pl.ds(0, 16)] = neg
        res_v[g, pl.ds(16, 16)] = neg

    b_lo = b_v[0, pl.ds(0, 16)]
    b_hi = b_v[0, pl.ds(16, 16)]

    @pl.loop(0, _RPW // 16)
    def _(i):
        base = i * 16
        bvec = batch_v[pl.ds(base, 16)]
        for j in range(16):
            r = base + j
            g = bvec[j]
            for half, bh in ((0, b_lo), (16, b_hi)):
                ds = pl.ds(half, 16)
                v = (dinv_v[r, ds]
                     * (a0_v[r, ds] + a1_v[r, ds] + t_v[r, ds]) + bh)
                v = jnp.where(v >= 0, v, 0.01 * v)
                h_v[r, ds] = v
                res_v[g, ds] = jnp.maximum(res_v[g, ds], v)

    pltpu.sync_copy(h_v, h_hbm.at[rows])
    pltpu.sync_copy(res_v, part_hbm.at[wid])


def _tc0_body(x_ref, w_ref, degp_ref, dinv_ref, t_ref):
    deg = degp_ref[0] + degp_ref[1] + 1.0
    dinv16 = lax.rsqrt(jnp.maximum(deg, 1.0))
    dinv = jnp.concatenate((dinv16, dinv16), axis=1)
    dinv_ref[...] = dinv
    hw = jnp.dot(x_ref[...], w_ref[...], preferred_element_type=jnp.float32)
    t_ref[...] = dinv * hw


def _tc_mid_body(acc_ref, t_ref, dinv_ref, b_ref, w_ref, tn_ref):
    dinv = dinv_ref[...]
    h = dinv * (acc_ref[0] + acc_ref[1] + t_ref[...]) + b_ref[...]
    h = jnp.where(h >= 0, h, 0.01 * h)
    hw = jnp.dot(h, w_ref[...], preferred_element_type=jnp.float32)
    tn_ref[...] = dinv * hw


def _tc_tail_body(part_ref, wout_ref, bout_ref, out_ref):
    pooled = jnp.max(part_ref[...], axis=0)[:_G]
    out_ref[...] = (
        jnp.dot(pooled, wout_ref[...], preferred_element_type=jnp.float32)
        + bout_ref[...]
    )


_tc0 = pl.pallas_call(
    _tc0_body,
    out_shape=[
        jax.ShapeDtypeStruct((_NPAD, _H), jnp.float32),
        jax.ShapeDtypeStruct((_NPAD, _H), jnp.float32),
    ],
)

_tc_mid = pl.pallas_call(
    _tc_mid_body,
    out_shape=jax.ShapeDtypeStruct((_NPAD, _H), jnp.float32),
)

_tc_tail = pl.pallas_call(
    _tc_tail_body,
    out_shape=jax.ShapeDtypeStruct((_G, 1), jnp.float32),
)


def kernel(x, edge_index, batch_index,
           W0, b0, W1, b1, W2, b2, W3, b3, W_out, b_out):
    src = edge_index[0]
    dst = edge_index[1]
    pad = _EPAD - _E
    srcp = jnp.concatenate(
        [src, jnp.zeros((pad,), jnp.int32)]).reshape(_NSUB, _NCT, _CHUNK)
    dstp = jnp.concatenate(
        [dst, jnp.full((pad,), _N, jnp.int32)]).reshape(_NSUB, _NCT, _CHUNK)
    ones = jnp.ones((_CHUNK, _DW), jnp.float32)
    xp = jnp.pad(x, ((0, _NPAD - _N), (0, 0)))
    batchp = jnp.concatenate(
        [batch_index, jnp.full((_NPAD - _N,), _G, jnp.int32)])

    degp = _degree_scatter(ones, dstp)
    dinv, t = _tc0(xp, W0, degp)
    for b, Wn in ((b0, W1), (b1, W2), (b2, W3)):
        acc = _edge_scatter_add(t, srcp, dstp)
        t = _tc_mid(acc, t, dinv, b.reshape(1, _H), Wn)
    acc = _edge_scatter_add(t, srcp, dstp)
    hid_pad, parts = _final_pool(acc, t, dinv, b3.reshape(1, _H), batchp)
    out = _tc_tail(parts, W_out, b_out.reshape(1, 1))
    return (out, hid_pad[:_N])
